# baseline (speedup 1.0000x reference)
import functools

import jax
import jax.numpy as jnp
from jax import lax
from jax.experimental import pallas as pl
from jax.experimental.pallas import tpu as pltpu
from jax.experimental.pallas import tpu_sc as plsc

N = 10000
E = 320000
D = 128

NC = 2
NS = 16
TILES = NC * NS

CHUNK = 128
CPT = 80
NCHUNKS = CPT * TILES
E_PAD = NCHUNKS * CHUNK
N_PAD = 10240
ROWS_PT = N_PAD // NS

BN = 640


def _prep_body(wm_ref, wr_ref, wu_ref, bm_ref, br_ref, bu_ref, wcat_ref, bcat_ref):
    wu1 = wu_ref[:D, :]
    f32 = jnp.float32
    wcat_ref[:, 0 * D:1 * D] = jnp.dot(wm_ref[:D, :], wu1, preferred_element_type=f32)
    wcat_ref[:, 1 * D:2 * D] = jnp.dot(wr_ref[:D, :], wu1, preferred_element_type=f32)
    wcat_ref[:, 2 * D:3 * D] = jnp.dot(wm_ref[D:, :], wu1, preferred_element_type=f32)
    wcat_ref[:, 3 * D:4 * D] = jnp.dot(wr_ref[D:, :], wu1, preferred_element_type=f32)
    wcat_ref[:, 4 * D:5 * D] = wu_ref[D:, :]
    bcat_ref[:, 0 * D:2 * D] = jnp.zeros((1, 2 * D), f32)
    bcat_ref[:, 2 * D:3 * D] = jnp.dot(bm_ref[...], wu1, preferred_element_type=f32)
    bcat_ref[:, 3 * D:4 * D] = jnp.dot(br_ref[...], wu1, preferred_element_type=f32)
    bcat_ref[:, 4 * D:5 * D] = bu_ref[...]


_prep = pl.pallas_call(
    _prep_body,
    out_shape=(
        jax.ShapeDtypeStruct((D, 5 * D), jnp.float32),
        jax.ShapeDtypeStruct((1, 5 * D), jnp.float32),
    ),
)


def _proj_body(x_ref, w_ref, b_ref, a_ref, c_ref, bt_ref, dt_ref, u_ref):
    p = jnp.dot(x_ref[...], w_ref[...], preferred_element_type=jnp.float32) + b_ref[...]
    a_ref[...] = p[:, 0 * D:1 * D]
    c_ref[...] = p[:, 1 * D:2 * D]
    bt_ref[...] = p[:, 2 * D:3 * D]
    dt_ref[...] = p[:, 3 * D:4 * D]
    u_ref[...] = p[:, 4 * D:5 * D]


_proj = pl.pallas_call(
    _proj_body,
    grid=(N_PAD // BN,),
    in_specs=[
        pl.BlockSpec((BN, D), lambda i: (i, 0)),
        pl.BlockSpec((D, 5 * D), lambda i: (0, 0)),
        pl.BlockSpec((1, 5 * D), lambda i: (0, 0)),
    ],
    out_specs=[pl.BlockSpec((BN, D), lambda i: (i, 0)) for _ in range(5)],
    out_shape=[jax.ShapeDtypeStruct((N_PAD, D), jnp.float32) for _ in range(5)],
)


def _sc_body(a_hbm, c_hbm, fidx_hbm, tidx_hbm, z2_hbm, z1_hbm,
             s_out, deg_out,
             sidx_v, didx_v, rows0_v, rows1_v, ones_v,
             acc_sp, indeg_sp, outdeg_sp, sem0, sem1):
    cid = lax.axis_index("c")
    sid = lax.axis_index("s")
    wid = cid * NS + sid
    row0 = sid * ROWS_PT

    pltpu.sync_copy(z2_hbm.at[pl.ds(row0, ROWS_PT)], acc_sp.at[pl.ds(row0, ROWS_PT)])
    pltpu.sync_copy(z1_hbm.at[pl.ds(row0, ROWS_PT)], indeg_sp.at[pl.ds(row0, ROWS_PT)])
    pltpu.sync_copy(z1_hbm.at[pl.ds(row0, ROWS_PT)], outdeg_sp.at[pl.ds(row0, ROWS_PT)])
    for j in range(CHUNK // 16):
        ones_v[pl.ds(j * 16, 16)] = jnp.ones((16,), jnp.float32)
    plsc.subcore_barrier()

    rows = (rows0_v, rows1_v)
    sems = (sem0, sem1)
    e0 = wid * (CPT * CHUNK)

    for tab_hbm, src_hbm, dst_hbm, deg_sp in (
        (a_hbm, fidx_hbm, tidx_hbm, indeg_sp),
        (c_hbm, tidx_hbm, fidx_hbm, outdeg_sp),
    ):
        pltpu.sync_copy(src_hbm.at[pl.ds(e0, CHUNK)], sidx_v.at[0])
        pltpu.sync_copy(dst_hbm.at[pl.ds(e0, CHUNK)], didx_v.at[0])
        pltpu.async_copy(tab_hbm.at[sidx_v.at[0]], rows[0], sems[0])

        def pair(i, carry, tab=tab_hbm, src=src_hbm, dst=dst_hbm, deg=deg_sp):
            for b in range(2):
                g = 2 * i + b

                @pl.when(g + 1 < CPT)
                def _prefetch():
                    off = e0 + (g + 1) * CHUNK
                    pltpu.sync_copy(src.at[pl.ds(off, CHUNK)], sidx_v.at[1 - b])
                    pltpu.sync_copy(dst.at[pl.ds(off, CHUNK)], didx_v.at[1 - b])
                    pltpu.async_copy(tab.at[sidx_v.at[1 - b]], rows[1 - b], sems[1 - b])

                pltpu.make_async_copy(tab.at[sidx_v.at[b]], rows[b], sems[b]).wait()
                pltpu.sync_copy(rows[b], acc_sp.at[didx_v.at[b]], add=True)
                pltpu.sync_copy(ones_v, deg.at[didx_v.at[b]], add=True)
            return carry

        lax.fori_loop(0, CPT // 2, pair, 0)

    plsc.subcore_barrier()

    pltpu.sync_copy(acc_sp.at[pl.ds(row0, ROWS_PT)], s_out.at[cid, pl.ds(row0, ROWS_PT)])
    pltpu.sync_copy(indeg_sp.at[pl.ds(row0, ROWS_PT)], deg_out.at[cid, 0, pl.ds(row0, ROWS_PT)])
    pltpu.sync_copy(outdeg_sp.at[pl.ds(row0, ROWS_PT)], deg_out.at[cid, 1, pl.ds(row0, ROWS_PT)])


_sc_scatter = functools.partial(
    pl.kernel,
    out_type=(
        jax.ShapeDtypeStruct((NC, N_PAD, D), jnp.float32),
        jax.ShapeDtypeStruct((NC, 2, N_PAD), jnp.float32),
    ),
    mesh=plsc.VectorSubcoreMesh(core_axis_name="c", subcore_axis_name="s"),
    scratch_types=[
        pltpu.VMEM((2, CHUNK), jnp.int32),
        pltpu.VMEM((2, CHUNK), jnp.int32),
        pltpu.VMEM((CHUNK, D), jnp.float32),
        pltpu.VMEM((CHUNK, D), jnp.float32),
        pltpu.VMEM((CHUNK,), jnp.float32),
        pltpu.VMEM_SHARED((N_PAD, D), jnp.float32),
        pltpu.VMEM_SHARED((N_PAD,), jnp.float32),
        pltpu.VMEM_SHARED((N_PAD,), jnp.float32),
        pltpu.SemaphoreType.DMA,
        pltpu.SemaphoreType.DMA,
    ],
)(_sc_body)


def _comb_body(s0_ref, s1_ref, u_ref, bt_ref, dt_ref, di_ref, do_ref, o_ref):
    o_ref[...] = (s0_ref[...] + s1_ref[...] + u_ref[...]
                  + di_ref[...] * bt_ref[...] + do_ref[...] * dt_ref[...])


_combine = pl.pallas_call(
    _comb_body,
    grid=(N_PAD // BN,),
    in_specs=[pl.BlockSpec((BN, D), lambda i: (i, 0)) for _ in range(5)]
    + [pl.BlockSpec((BN, 1), lambda i: (i, 0)) for _ in range(2)],
    out_specs=pl.BlockSpec((BN, D), lambda i: (i, 0)),
    out_shape=jax.ShapeDtypeStruct((N_PAD, D), jnp.float32),
)


def kernel(node_states, from_idx, to_idx, W_msg, b_msg, W_rmsg, b_rmsg, W_upd, b_upd):
    wcat, bcat = _prep(W_msg, W_rmsg, W_upd,
                       b_msg.reshape(1, D), b_rmsg.reshape(1, D), b_upd.reshape(1, D))
    ns_pad = jnp.pad(node_states, ((0, N_PAD - N), (0, 0)))
    a, c, bt, dt, u = _proj(ns_pad, wcat, bcat)
    idx_pad = jnp.full((E_PAD - E,), N, jnp.int32)
    fi = jnp.concatenate([from_idx, idx_pad])
    ti = jnp.concatenate([to_idx, idx_pad])
    z2 = jnp.zeros((N_PAD, D), jnp.float32)
    z1 = jnp.zeros((N_PAD,), jnp.float32)
    s, degs = _sc_scatter(a, c, fi, ti, z2, z1)
    indeg = (degs[0, 0] + degs[1, 0]).reshape(N_PAD, 1)
    outdeg = (degs[0, 1] + degs[1, 1]).reshape(N_PAD, 1)
    return _combine(s[0], s[1], u, bt, dt, indeg, outdeg)[:N]


# speedup vs baseline: 1.0063x; 1.0063x over previous
import functools

import jax
import jax.numpy as jnp
from jax import lax
from jax.experimental import pallas as pl
from jax.experimental.pallas import tpu as pltpu
from jax.experimental.pallas import tpu_sc as plsc

N = 10000
E = 320000
D = 128

NC = 2
NS = 16
TILES = NC * NS

CHUNK = 128
CPT = 80
NCHUNKS = CPT * TILES
E_PAD = NCHUNKS * CHUNK
N_PAD = 10240
ROWS_PT = N_PAD // NS

BN = 640


def _prep_body(wm_ref, wr_ref, wu_ref, bm_ref, br_ref, bu_ref, wcat_ref, bcat_ref):
    wu1 = wu_ref[:D, :]
    f32 = jnp.float32
    wcat_ref[:, 0 * D:1 * D] = jnp.dot(wm_ref[:D, :], wu1, preferred_element_type=f32)
    wcat_ref[:, 1 * D:2 * D] = jnp.dot(wr_ref[:D, :], wu1, preferred_element_type=f32)
    wcat_ref[:, 2 * D:3 * D] = jnp.dot(wm_ref[D:, :], wu1, preferred_element_type=f32)
    wcat_ref[:, 3 * D:4 * D] = jnp.dot(wr_ref[D:, :], wu1, preferred_element_type=f32)
    wcat_ref[:, 4 * D:5 * D] = wu_ref[D:, :]
    bcat_ref[:, 0 * D:2 * D] = jnp.zeros((1, 2 * D), f32)
    bcat_ref[:, 2 * D:3 * D] = jnp.dot(bm_ref[...], wu1, preferred_element_type=f32)
    bcat_ref[:, 3 * D:4 * D] = jnp.dot(br_ref[...], wu1, preferred_element_type=f32)
    bcat_ref[:, 4 * D:5 * D] = bu_ref[...]


_prep = pl.pallas_call(
    _prep_body,
    out_shape=(
        jax.ShapeDtypeStruct((D, 5 * D), jnp.float32),
        jax.ShapeDtypeStruct((1, 5 * D), jnp.float32),
    ),
)


def _proj_body(x_ref, w_ref, b_ref, a_ref, c_ref, bt_ref, dt_ref, u_ref):
    p = jnp.dot(x_ref[...], w_ref[...], preferred_element_type=jnp.float32) + b_ref[...]
    a_ref[...] = p[:, 0 * D:1 * D]
    c_ref[...] = p[:, 1 * D:2 * D]
    bt_ref[...] = p[:, 2 * D:3 * D]
    dt_ref[...] = p[:, 3 * D:4 * D]
    u_ref[...] = p[:, 4 * D:5 * D]


_proj = pl.pallas_call(
    _proj_body,
    grid=(N_PAD // BN,),
    in_specs=[
        pl.BlockSpec((BN, D), lambda i: (i, 0)),
        pl.BlockSpec((D, 5 * D), lambda i: (0, 0)),
        pl.BlockSpec((1, 5 * D), lambda i: (0, 0)),
    ],
    out_specs=[pl.BlockSpec((BN, D), lambda i: (i, 0)) for _ in range(5)],
    out_shape=[jax.ShapeDtypeStruct((N_PAD, D), jnp.float32) for _ in range(5)],
)


def _sc_body(a_hbm, c_hbm, fidx_hbm, tidx_hbm, z2_hbm, z1_hbm,
             s_out, deg_out,
             sidx_v, didx_v, rows0_v, rows1_v, ones_v,
             acc_sp, indeg_sp, outdeg_sp, sem0, sem1,
             si0, si1, sd0, sd1):
    cid = lax.axis_index("c")
    sid = lax.axis_index("s")
    wid = cid * NS + sid
    row0 = sid * ROWS_PT

    pltpu.sync_copy(z2_hbm.at[pl.ds(row0, ROWS_PT)], acc_sp.at[pl.ds(row0, ROWS_PT)])
    pltpu.sync_copy(z1_hbm.at[pl.ds(row0, ROWS_PT)], indeg_sp.at[pl.ds(row0, ROWS_PT)])
    pltpu.sync_copy(z1_hbm.at[pl.ds(row0, ROWS_PT)], outdeg_sp.at[pl.ds(row0, ROWS_PT)])
    for j in range(CHUNK // 16):
        ones_v[pl.ds(j * 16, 16)] = jnp.ones((16,), jnp.float32)
    plsc.subcore_barrier()

    rows = (rows0_v, rows1_v)
    sems = (sem0, sem1)
    sis = (si0, si1)
    sds = (sd0, sd1)
    e0 = wid * (CPT * CHUNK)

    for tab_hbm, src_hbm, dst_hbm, deg_sp in (
        (a_hbm, fidx_hbm, tidx_hbm, indeg_sp),
        (c_hbm, tidx_hbm, fidx_hbm, outdeg_sp),
    ):
        for g in range(2):
            off = e0 + g * CHUNK
            pltpu.async_copy(src_hbm.at[pl.ds(off, CHUNK)], sidx_v.at[g], sis[g])
            pltpu.async_copy(dst_hbm.at[pl.ds(off, CHUNK)], didx_v.at[g], sds[g])
        pltpu.make_async_copy(src_hbm.at[pl.ds(e0, CHUNK)], sidx_v.at[0], sis[0]).wait()
        pltpu.async_copy(tab_hbm.at[sidx_v.at[0]], rows[0], sems[0])

        def pair(i, carry, tab=tab_hbm, src=src_hbm, dst=dst_hbm, deg=deg_sp):
            for b in range(2):
                g = 2 * i + b

                @pl.when(g + 1 < CPT)
                def _gather_next():
                    off = e0 + (g + 1) * CHUNK
                    pltpu.make_async_copy(
                        src.at[pl.ds(off, CHUNK)], sidx_v.at[1 - b], sis[1 - b]).wait()
                    pltpu.async_copy(tab.at[sidx_v.at[1 - b]], rows[1 - b], sems[1 - b])

                pltpu.make_async_copy(tab.at[sidx_v.at[b]], rows[b], sems[b]).wait()
                pltpu.make_async_copy(
                    dst.at[pl.ds(e0 + g * CHUNK, CHUNK)], didx_v.at[b], sds[b]).wait()
                pltpu.sync_copy(rows[b], acc_sp.at[didx_v.at[b]], add=True)
                pltpu.sync_copy(ones_v, deg.at[didx_v.at[b]], add=True)

                @pl.when(g + 2 < CPT)
                def _idx_next():
                    off = e0 + (g + 2) * CHUNK
                    pltpu.async_copy(src.at[pl.ds(off, CHUNK)], sidx_v.at[b], sis[b])
                    pltpu.async_copy(dst.at[pl.ds(off, CHUNK)], didx_v.at[b], sds[b])
            return carry

        lax.fori_loop(0, CPT // 2, pair, 0)

    plsc.subcore_barrier()

    pltpu.sync_copy(acc_sp.at[pl.ds(row0, ROWS_PT)], s_out.at[cid, pl.ds(row0, ROWS_PT)])
    pltpu.sync_copy(indeg_sp.at[pl.ds(row0, ROWS_PT)], deg_out.at[cid, 0, pl.ds(row0, ROWS_PT)])
    pltpu.sync_copy(outdeg_sp.at[pl.ds(row0, ROWS_PT)], deg_out.at[cid, 1, pl.ds(row0, ROWS_PT)])


_sc_scatter = functools.partial(
    pl.kernel,
    out_type=(
        jax.ShapeDtypeStruct((NC, N_PAD, D), jnp.float32),
        jax.ShapeDtypeStruct((NC, 2, N_PAD), jnp.float32),
    ),
    mesh=plsc.VectorSubcoreMesh(core_axis_name="c", subcore_axis_name="s"),
    scratch_types=[
        pltpu.VMEM((2, CHUNK), jnp.int32),
        pltpu.VMEM((2, CHUNK), jnp.int32),
        pltpu.VMEM((CHUNK, D), jnp.float32),
        pltpu.VMEM((CHUNK, D), jnp.float32),
        pltpu.VMEM((CHUNK,), jnp.float32),
        pltpu.VMEM_SHARED((N_PAD, D), jnp.float32),
        pltpu.VMEM_SHARED((N_PAD,), jnp.float32),
        pltpu.VMEM_SHARED((N_PAD,), jnp.float32),
        pltpu.SemaphoreType.DMA,
        pltpu.SemaphoreType.DMA,
        pltpu.SemaphoreType.DMA,
        pltpu.SemaphoreType.DMA,
        pltpu.SemaphoreType.DMA,
        pltpu.SemaphoreType.DMA,
    ],
)(_sc_body)


def _comb_body(s0_ref, s1_ref, u_ref, bt_ref, dt_ref, di_ref, do_ref, o_ref):
    o_ref[...] = (s0_ref[...] + s1_ref[...] + u_ref[...]
                  + di_ref[...] * bt_ref[...] + do_ref[...] * dt_ref[...])


_combine = pl.pallas_call(
    _comb_body,
    grid=(N_PAD // BN,),
    in_specs=[pl.BlockSpec((BN, D), lambda i: (i, 0)) for _ in range(5)]
    + [pl.BlockSpec((BN, 1), lambda i: (i, 0)) for _ in range(2)],
    out_specs=pl.BlockSpec((BN, D), lambda i: (i, 0)),
    out_shape=jax.ShapeDtypeStruct((N_PAD, D), jnp.float32),
)


def kernel(node_states, from_idx, to_idx, W_msg, b_msg, W_rmsg, b_rmsg, W_upd, b_upd):
    wcat, bcat = _prep(W_msg, W_rmsg, W_upd,
                       b_msg.reshape(1, D), b_rmsg.reshape(1, D), b_upd.reshape(1, D))
    ns_pad = jnp.pad(node_states, ((0, N_PAD - N), (0, 0)))
    a, c, bt, dt, u = _proj(ns_pad, wcat, bcat)
    idx_pad = jnp.full((E_PAD - E,), N, jnp.int32)
    fi = jnp.concatenate([from_idx, idx_pad])
    ti = jnp.concatenate([to_idx, idx_pad])
    z2 = jnp.zeros((N_PAD, D), jnp.float32)
    z1 = jnp.zeros((N_PAD,), jnp.float32)
    s, degs = _sc_scatter(a, c, fi, ti, z2, z1)
    indeg = (degs[0, 0] + degs[1, 0]).reshape(N_PAD, 1)
    outdeg = (degs[0, 1] + degs[1, 1]).reshape(N_PAD, 1)
    return _combine(s[0], s[1], u, bt, dt, indeg, outdeg)[:N]


# speedup vs baseline: 2.8193x; 2.8017x over previous
import functools

import jax
import jax.numpy as jnp
from jax import lax
from jax.experimental import pallas as pl
from jax.experimental.pallas import tpu as pltpu
from jax.experimental.pallas import tpu_sc as plsc

N = 10000
E = 320000
D = 128

NC = 2
NS = 16
TILES = NC * NS

CHUNK = 128
CPT = 80
NCHUNKS = CPT * TILES
E_PAD = NCHUNKS * CHUNK
N_PAD = 10240
ROWS_PT = N_PAD // NS

BN = 640


def _prep_body(wm_ref, wr_ref, wu_ref, bm_ref, br_ref, bu_ref, wcat_ref, bcat_ref):
    wu1 = wu_ref[:D, :]
    f32 = jnp.float32
    wcat_ref[:, 0 * D:1 * D] = jnp.dot(wm_ref[:D, :], wu1, preferred_element_type=f32)
    wcat_ref[:, 1 * D:2 * D] = jnp.dot(wr_ref[:D, :], wu1, preferred_element_type=f32)
    wcat_ref[:, 2 * D:3 * D] = jnp.dot(wm_ref[D:, :], wu1, preferred_element_type=f32)
    wcat_ref[:, 3 * D:4 * D] = jnp.dot(wr_ref[D:, :], wu1, preferred_element_type=f32)
    wcat_ref[:, 4 * D:5 * D] = wu_ref[D:, :]
    bcat_ref[:, 0 * D:2 * D] = jnp.zeros((1, 2 * D), f32)
    bcat_ref[:, 2 * D:3 * D] = jnp.dot(bm_ref[...], wu1, preferred_element_type=f32)
    bcat_ref[:, 3 * D:4 * D] = jnp.dot(br_ref[...], wu1, preferred_element_type=f32)
    bcat_ref[:, 4 * D:5 * D] = bu_ref[...]


_prep = pl.pallas_call(
    _prep_body,
    out_shape=(
        jax.ShapeDtypeStruct((D, 5 * D), jnp.float32),
        jax.ShapeDtypeStruct((1, 5 * D), jnp.float32),
    ),
)


def _proj_body(x_ref, w_ref, b_ref, a_ref, c_ref, bt_ref, dt_ref, u_ref):
    p = jnp.dot(x_ref[...], w_ref[...], preferred_element_type=jnp.float32) + b_ref[...]
    a_ref[...] = p[:, 0 * D:1 * D]
    c_ref[...] = p[:, 1 * D:2 * D]
    bt_ref[...] = p[:, 2 * D:3 * D]
    dt_ref[...] = p[:, 3 * D:4 * D]
    u_ref[...] = p[:, 4 * D:5 * D]


_proj = pl.pallas_call(
    _proj_body,
    grid=(N_PAD // BN,),
    in_specs=[
        pl.BlockSpec((BN, D), lambda i: (i, 0)),
        pl.BlockSpec((D, 5 * D), lambda i: (0, 0)),
        pl.BlockSpec((1, 5 * D), lambda i: (0, 0)),
    ],
    out_specs=[pl.BlockSpec((BN, D), lambda i: (i, 0)) for _ in range(5)],
    out_shape=[jax.ShapeDtypeStruct((N_PAD, D), jnp.float32) for _ in range(5)],
)


def _sc_body(a_hbm, c_hbm, fidx_hbm, tidx_hbm, z2_hbm, z1_hbm,
             s_out, deg_out,
             sidx_v, didx_v, rows0_v, rows1_v, ones_v,
             acc_sp, indeg_sp, outdeg_sp, sem0, sem1,
             si0, si1, sd0, sd1):
    cid = lax.axis_index("c")
    sid = lax.axis_index("s")
    wid = cid * NS + sid
    row0 = sid * ROWS_PT

    pltpu.sync_copy(z2_hbm.at[pl.ds(row0, ROWS_PT)], acc_sp.at[pl.ds(row0, ROWS_PT)])
    pltpu.sync_copy(z1_hbm.at[pl.ds(row0, ROWS_PT)], indeg_sp.at[pl.ds(row0, ROWS_PT)])
    pltpu.sync_copy(z1_hbm.at[pl.ds(row0, ROWS_PT)], outdeg_sp.at[pl.ds(row0, ROWS_PT)])
    for j in range(CHUNK // 16):
        ones_v[pl.ds(j * 16, 16)] = jnp.ones((16,), jnp.float32)
    plsc.subcore_barrier()

    rows = (rows0_v, rows1_v)
    sems = (sem0, sem1)
    sis = (si0, si1)
    sds = (sd0, sd1)
    e0 = wid * (CPT * CHUNK)

    for tab_hbm, src_hbm, dst_hbm, deg_sp in (
        (a_hbm, fidx_hbm, tidx_hbm, indeg_sp),
        (c_hbm, tidx_hbm, fidx_hbm, outdeg_sp),
    ):
        for g in range(2):
            off = e0 + g * CHUNK
            pltpu.async_copy(src_hbm.at[pl.ds(off, CHUNK)], sidx_v.at[g], sis[g])
            pltpu.async_copy(dst_hbm.at[pl.ds(off, CHUNK)], didx_v.at[g], sds[g])
        pltpu.make_async_copy(src_hbm.at[pl.ds(e0, CHUNK)], sidx_v.at[0], sis[0]).wait()
        pltpu.async_copy(tab_hbm.at[sidx_v.at[0]], rows[0], sems[0])

        def pair(i, carry, tab=tab_hbm, src=src_hbm, dst=dst_hbm, deg=deg_sp):
            for b in range(2):
                g = 2 * i + b

                @pl.when(g + 1 < CPT)
                def _gather_next():
                    off = e0 + (g + 1) * CHUNK
                    pltpu.make_async_copy(
                        src.at[pl.ds(off, CHUNK)], sidx_v.at[1 - b], sis[1 - b]).wait()
                    pltpu.async_copy(tab.at[sidx_v.at[1 - b]], rows[1 - b], sems[1 - b])

                pltpu.make_async_copy(tab.at[sidx_v.at[b]], rows[b], sems[b]).wait()
                pltpu.make_async_copy(
                    dst.at[pl.ds(e0 + g * CHUNK, CHUNK)], didx_v.at[b], sds[b]).wait()
                pltpu.sync_copy(rows[b], acc_sp.at[didx_v.at[b]], add=True)
                pltpu.sync_copy(ones_v, deg.at[didx_v.at[b]], add=True)

                @pl.when(g + 2 < CPT)
                def _idx_next():
                    off = e0 + (g + 2) * CHUNK
                    pltpu.async_copy(src.at[pl.ds(off, CHUNK)], sidx_v.at[b], sis[b])
                    pltpu.async_copy(dst.at[pl.ds(off, CHUNK)], didx_v.at[b], sds[b])
            return carry

        lax.fori_loop(0, CPT // 2, pair, 0)

    plsc.subcore_barrier()

    pltpu.sync_copy(acc_sp.at[pl.ds(row0, ROWS_PT)], s_out.at[cid, pl.ds(row0, ROWS_PT)])
    pltpu.sync_copy(indeg_sp.at[pl.ds(row0, ROWS_PT)], deg_out.at[cid, 0, pl.ds(row0, ROWS_PT)])
    pltpu.sync_copy(outdeg_sp.at[pl.ds(row0, ROWS_PT)], deg_out.at[cid, 1, pl.ds(row0, ROWS_PT)])


_sc_scatter = functools.partial(
    pl.kernel,
    out_type=(
        jax.ShapeDtypeStruct((NC, N_PAD, D), jnp.float32),
        jax.ShapeDtypeStruct((NC, 2, N_PAD), jnp.float32),
    ),
    mesh=plsc.VectorSubcoreMesh(core_axis_name="c", subcore_axis_name="s"),
    scratch_types=[
        pltpu.VMEM((2, CHUNK), jnp.int32),
        pltpu.VMEM((2, CHUNK), jnp.int32),
        pltpu.VMEM((CHUNK, D), jnp.float32),
        pltpu.VMEM((CHUNK, D), jnp.float32),
        pltpu.VMEM((CHUNK,), jnp.float32),
        pltpu.VMEM_SHARED((N_PAD, D), jnp.float32),
        pltpu.VMEM_SHARED((N_PAD,), jnp.float32),
        pltpu.VMEM_SHARED((N_PAD,), jnp.float32),
        pltpu.SemaphoreType.DMA,
        pltpu.SemaphoreType.DMA,
        pltpu.SemaphoreType.DMA,
        pltpu.SemaphoreType.DMA,
        pltpu.SemaphoreType.DMA,
        pltpu.SemaphoreType.DMA,
    ],
)(_sc_body)


def _comb_body(s0_ref, s1_ref, u_ref, bt_ref, dt_ref, di_ref, do_ref, o_ref):
    o_ref[...] = (s0_ref[...] + s1_ref[...] + u_ref[...]
                  + di_ref[...] * bt_ref[...] + do_ref[...] * dt_ref[...])


_combine = pl.pallas_call(
    _comb_body,
    grid=(N_PAD // BN,),
    in_specs=[pl.BlockSpec((BN, D), lambda i: (i, 0)) for _ in range(5)]
    + [pl.BlockSpec((BN, 1), lambda i: (i, 0)) for _ in range(2)],
    out_specs=pl.BlockSpec((BN, D), lambda i: (i, 0)),
    out_shape=jax.ShapeDtypeStruct((N_PAD, D), jnp.float32),
)


def kernel(node_states, from_idx, to_idx, W_msg, b_msg, W_rmsg, b_rmsg, W_upd, b_upd):
    wcat, bcat = _prep(W_msg, W_rmsg, W_upd,
                       b_msg.reshape(1, D), b_rmsg.reshape(1, D), b_upd.reshape(1, D))
    ns_pad = jnp.pad(node_states, ((0, N_PAD - N), (0, 0)))
    a, c, bt, dt, u = _proj(ns_pad, wcat, bcat)
    idx_pad = N + (jnp.arange(E_PAD - E, dtype=jnp.int32) % (N_PAD - N))
    fi = jnp.concatenate([from_idx, idx_pad])
    ti = jnp.concatenate([to_idx, idx_pad])
    z2 = jnp.zeros((N_PAD, D), jnp.float32)
    z1 = jnp.zeros((N_PAD,), jnp.float32)
    s, degs = _sc_scatter(a, c, fi, ti, z2, z1)
    indeg = (degs[0, 0] + degs[1, 0]).reshape(N_PAD, 1)
    outdeg = (degs[0, 1] + degs[1, 1]).reshape(N_PAD, 1)
    return _combine(s[0], s[1], u, bt, dt, indeg, outdeg)[:N]


# speedup vs baseline: 3.0518x; 1.0825x over previous
import functools

import jax
import jax.numpy as jnp
from jax import lax
from jax.experimental import pallas as pl
from jax.experimental.pallas import tpu as pltpu
from jax.experimental.pallas import tpu_sc as plsc

N = 10000
E = 320000
D = 128

NC = 2
NS = 16
TILES = NC * NS

CHUNK = 128
CPT = 80
NCHUNKS = CPT * TILES
E_PAD = NCHUNKS * CHUNK
HALF = CPT // 2
N_PAD = 10240
ROWS_PT = N_PAD // NS

BN = 640


def _prep_body(wm_ref, wr_ref, wu_ref, bm_ref, br_ref, bu_ref, wcat_ref, bcat_ref):
    wu1 = wu_ref[:D, :]
    f32 = jnp.float32
    wcat_ref[:, 0 * D:1 * D] = jnp.dot(wm_ref[:D, :], wu1, preferred_element_type=f32)
    wcat_ref[:, 1 * D:2 * D] = jnp.dot(wr_ref[:D, :], wu1, preferred_element_type=f32)
    wcat_ref[:, 2 * D:3 * D] = jnp.dot(wm_ref[D:, :], wu1, preferred_element_type=f32)
    wcat_ref[:, 3 * D:4 * D] = jnp.dot(wr_ref[D:, :], wu1, preferred_element_type=f32)
    wcat_ref[:, 4 * D:5 * D] = wu_ref[D:, :]
    bcat_ref[:, 0 * D:2 * D] = jnp.zeros((1, 2 * D), f32)
    bcat_ref[:, 2 * D:3 * D] = jnp.dot(bm_ref[...], wu1, preferred_element_type=f32)
    bcat_ref[:, 3 * D:4 * D] = jnp.dot(br_ref[...], wu1, preferred_element_type=f32)
    bcat_ref[:, 4 * D:5 * D] = bu_ref[...]


_prep = pl.pallas_call(
    _prep_body,
    out_shape=(
        jax.ShapeDtypeStruct((D, 5 * D), jnp.float32),
        jax.ShapeDtypeStruct((1, 5 * D), jnp.float32),
    ),
)


def _proj_body(x_ref, w_ref, b_ref, a_ref, c_ref, bt_ref, dt_ref, u_ref):
    p = jnp.dot(x_ref[...], w_ref[...], preferred_element_type=jnp.float32) + b_ref[...]
    a_ref[...] = p[:, 0 * D:1 * D]
    c_ref[...] = p[:, 1 * D:2 * D]
    bt_ref[...] = p[:, 2 * D:3 * D]
    dt_ref[...] = p[:, 3 * D:4 * D]
    u_ref[...] = p[:, 4 * D:5 * D]


_proj = pl.pallas_call(
    _proj_body,
    grid=(N_PAD // BN,),
    in_specs=[
        pl.BlockSpec((BN, D), lambda i: (i, 0)),
        pl.BlockSpec((D, 5 * D), lambda i: (0, 0)),
        pl.BlockSpec((1, 5 * D), lambda i: (0, 0)),
    ],
    out_specs=[pl.BlockSpec((BN, D), lambda i: (i, 0)) for _ in range(5)],
    out_shape=[jax.ShapeDtypeStruct((N_PAD, D), jnp.float32) for _ in range(5)],
)


def _sc_body(a_hbm, c_hbm, fidx_hbm, tidx_hbm, z2_hbm, z1_hbm,
             s_out, deg_out,
             sidx_v, didx_v, rows0_v, rows1_v, ones_v,
             acc_sp, indeg_sp, outdeg_sp, sem0, sem1):
    cid = lax.axis_index("c")
    sid = lax.axis_index("s")
    wid = cid * NS + sid
    row0 = sid * ROWS_PT

    pltpu.sync_copy(z2_hbm.at[pl.ds(row0, ROWS_PT)], acc_sp.at[pl.ds(row0, ROWS_PT)])
    pltpu.sync_copy(z1_hbm.at[pl.ds(row0, ROWS_PT)], indeg_sp.at[pl.ds(row0, ROWS_PT)])
    pltpu.sync_copy(z1_hbm.at[pl.ds(row0, ROWS_PT)], outdeg_sp.at[pl.ds(row0, ROWS_PT)])
    for j in range(CHUNK // 16):
        ones_v[pl.ds(j * 16, 16)] = jnp.ones((16,), jnp.float32)
    plsc.subcore_barrier()

    rows = (rows0_v, rows1_v)
    sems = (sem0, sem1)
    e0 = wid * (CPT * CHUNK)

    for tab_hbm, src_hbm, dst_hbm, deg_sp in (
        (a_hbm, fidx_hbm, tidx_hbm, indeg_sp),
        (c_hbm, tidx_hbm, fidx_hbm, outdeg_sp),
    ):
        for h in range(2):
            off0 = e0 + h * (HALF * CHUNK)
            pltpu.sync_copy(src_hbm.at[pl.ds(off0, HALF * CHUNK)], sidx_v)
            pltpu.sync_copy(dst_hbm.at[pl.ds(off0, HALF * CHUNK)], didx_v)
            pltpu.async_copy(tab_hbm.at[sidx_v.at[pl.ds(0, CHUNK)]], rows[0], sems[0])

            def pair(i, carry, tab=tab_hbm, deg=deg_sp):
                for b in range(2):
                    g = 2 * i + b

                    @pl.when(g + 1 < HALF)
                    def _gather_next():
                        idx = sidx_v.at[pl.ds((g + 1) * CHUNK, CHUNK)]
                        pltpu.async_copy(tab.at[idx], rows[1 - b], sems[1 - b])

                    pltpu.make_async_copy(
                        tab.at[sidx_v.at[pl.ds(g * CHUNK, CHUNK)]], rows[b], sems[b]).wait()
                    dst = didx_v.at[pl.ds(g * CHUNK, CHUNK)]
                    pltpu.sync_copy(rows[b], acc_sp.at[dst], add=True)
                    pltpu.sync_copy(ones_v, deg.at[dst], add=True)
                return carry

            lax.fori_loop(0, HALF // 2, pair, 0)

    plsc.subcore_barrier()

    pltpu.sync_copy(acc_sp.at[pl.ds(row0, ROWS_PT)], s_out.at[cid, pl.ds(row0, ROWS_PT)])
    pltpu.sync_copy(indeg_sp.at[pl.ds(row0, ROWS_PT)], deg_out.at[cid, 0, pl.ds(row0, ROWS_PT)])
    pltpu.sync_copy(outdeg_sp.at[pl.ds(row0, ROWS_PT)], deg_out.at[cid, 1, pl.ds(row0, ROWS_PT)])


_sc_scatter = functools.partial(
    pl.kernel,
    out_type=(
        jax.ShapeDtypeStruct((NC, N_PAD, D), jnp.float32),
        jax.ShapeDtypeStruct((NC, 2, N_PAD), jnp.float32),
    ),
    mesh=plsc.VectorSubcoreMesh(core_axis_name="c", subcore_axis_name="s"),
    scratch_types=[
        pltpu.VMEM((HALF * CHUNK,), jnp.int32),
        pltpu.VMEM((HALF * CHUNK,), jnp.int32),
        pltpu.VMEM((CHUNK, D), jnp.float32),
        pltpu.VMEM((CHUNK, D), jnp.float32),
        pltpu.VMEM((CHUNK,), jnp.float32),
        pltpu.VMEM_SHARED((N_PAD, D), jnp.float32),
        pltpu.VMEM_SHARED((N_PAD,), jnp.float32),
        pltpu.VMEM_SHARED((N_PAD,), jnp.float32),
        pltpu.SemaphoreType.DMA,
        pltpu.SemaphoreType.DMA,
    ],
)(_sc_body)


def _comb_body(s0_ref, s1_ref, u_ref, bt_ref, dt_ref, di_ref, do_ref, o_ref):
    o_ref[...] = (s0_ref[...] + s1_ref[...] + u_ref[...]
                  + di_ref[...] * bt_ref[...] + do_ref[...] * dt_ref[...])


_combine = pl.pallas_call(
    _comb_body,
    grid=(N_PAD // BN,),
    in_specs=[pl.BlockSpec((BN, D), lambda i: (i, 0)) for _ in range(5)]
    + [pl.BlockSpec((BN, 1), lambda i: (i, 0)) for _ in range(2)],
    out_specs=pl.BlockSpec((BN, D), lambda i: (i, 0)),
    out_shape=jax.ShapeDtypeStruct((N_PAD, D), jnp.float32),
)


def kernel(node_states, from_idx, to_idx, W_msg, b_msg, W_rmsg, b_rmsg, W_upd, b_upd):
    wcat, bcat = _prep(W_msg, W_rmsg, W_upd,
                       b_msg.reshape(1, D), b_rmsg.reshape(1, D), b_upd.reshape(1, D))
    ns_pad = jnp.pad(node_states, ((0, N_PAD - N), (0, 0)))
    a, c, bt, dt, u = _proj(ns_pad, wcat, bcat)
    idx_pad = N + (jnp.arange(E_PAD - E, dtype=jnp.int32) % (N_PAD - N))
    fi = jnp.concatenate([from_idx, idx_pad])
    ti = jnp.concatenate([to_idx, idx_pad])
    z2 = jnp.zeros((N_PAD, D), jnp.float32)
    z1 = jnp.zeros((N_PAD,), jnp.float32)
    s, degs = _sc_scatter(a, c, fi, ti, z2, z1)
    indeg = (degs[0, 0] + degs[1, 0]).reshape(N_PAD, 1)
    outdeg = (degs[0, 1] + degs[1, 1]).reshape(N_PAD, 1)
    return _combine(s[0], s[1], u, bt, dt, indeg, outdeg)[:N]
